# Initial kernel scaffold; baseline (speedup 1.0000x reference)
import jax
import jax.numpy as jnp
from jax.experimental import pallas as pl


def kernel(pc1, pc2, feature1, feature2, W0, b0, W1, b1, W2, b2, wn1_Wa, wn1_ba, wn1_gamma, wn1_beta, wn1_Wb, wn1_bb, wn2_Wa, wn2_ba, wn2_gamma, wn2_beta, wn2_Wb, wn2_bb):
    raise NotImplementedError("write your pallas kernel here")


# speedup vs baseline: 13.1980x; 13.1980x over previous
import functools

import jax
import jax.numpy as jnp
from jax import lax
from jax.experimental import pallas as pl
from jax.experimental.pallas import tpu as pltpu
from jax.experimental.pallas import tpu_sc as plsc

_B, _N, _C, _K = 2, 4096, 256, 16
_BN = _B * _N
_M = _BN * _K
_W = _C + 128
_F32 = jnp.float32


def _prep_body(pc1p, pc2p, pc1T, pc2T, pc2w, f1, f2, W0f1, W0f2, W0d8, b0,
               aq, bk2, bk1, y1p, t1):
    x1 = pc1p[...]
    x2 = pc2p[...]
    n1 = jnp.sum(x1 * x1, axis=1, keepdims=True)
    li = lax.broadcasted_iota(jnp.int32, x1.shape, 1)
    aq[...] = jnp.where(li < 3, x1, jnp.where(li == 4, n1, 0.0))
    for xt, bk in ((pc1T, bk1), (pc2T, bk2)):
        xv = xt[...]
        nn = (xv[0:1, :] * xv[0:1, :] + xv[1:2, :] * xv[1:2, :]
              + xv[2:3, :] * xv[2:3, :])
        si = lax.broadcasted_iota(jnp.int32, xv.shape, 0)
        bk[...] = jnp.where(si < 3, xv, jnp.where(si == 3, nn, 0.0))
    y1p[...] = (jnp.dot(f1[...], W0f1[...], preferred_element_type=_F32) + b0[...]
                - jnp.dot(x1, W0d8[...], preferred_element_type=_F32))
    feat = (jnp.dot(f2[...], W0f2[...], preferred_element_type=_F32)
            + jnp.dot(x2, W0d8[...], preferred_element_type=_F32))
    t1[...] = jnp.concatenate([feat, pc2w[...]], axis=1)


def _prep(pc1p8, pc2p8, pc1T8, pc2T8, pc2w, f1, f2, W0f1, W0f2, W0d8, b0r):
    T = 512
    grid = (_BN // T,)
    row = lambda i: (i, 0)
    col = lambda i: (0, i)
    const = lambda i: (0, 0)
    return pl.pallas_call(
        _prep_body,
        grid=grid,
        in_specs=[pl.BlockSpec((T, 8), row), pl.BlockSpec((T, 8), row),
                  pl.BlockSpec((8, T), col), pl.BlockSpec((8, T), col),
                  pl.BlockSpec((T, 128), row),
                  pl.BlockSpec((T, _C), row), pl.BlockSpec((T, _C), row),
                  pl.BlockSpec((_C, _C), const), pl.BlockSpec((_C, _C), const),
                  pl.BlockSpec((8, _C), const), pl.BlockSpec((1, _C), const)],
        out_specs=[pl.BlockSpec((T, 8), row), pl.BlockSpec((8, T), col),
                   pl.BlockSpec((8, T), col),
                   pl.BlockSpec((T, _C), row), pl.BlockSpec((T, _W), row)],
        out_shape=[jax.ShapeDtypeStruct((_BN, 8), _F32),
                   jax.ShapeDtypeStruct((8, _BN), _F32),
                   jax.ShapeDtypeStruct((8, _BN), _F32),
                   jax.ShapeDtypeStruct((_BN, _C), _F32),
                   jax.ShapeDtypeStruct((_BN, _W), _F32)],
    )(pc1p8, pc2p8, pc1T8, pc2T8, pc2w, f1, f2, W0f1, W0f2, W0d8, b0r)


_TQ = 256


def _knn_body(aq, bk, idx_out):
    q = aq[0]
    ks = bk[0]
    li = lax.broadcasted_iota(jnp.int32, q.shape, 1)
    si = lax.broadcasted_iota(jnp.int32, ks.shape, 0)
    qm = jnp.where(li < 3, q, 0.0)
    km = jnp.where(si < 3, ks, 0.0)
    s = lax.dot_general(qm, km, (((1,), (0,)), ((), ())),
                        preferred_element_type=_F32)
    d = (q[:, 4:5] + ks[3:4, :]) - 2.0 * s
    b = pl.program_id(0)
    iota = lax.broadcasted_iota(jnp.int32, d.shape, 1)
    cols = []
    for _ in range(_K):
        mval = jnp.min(d, axis=1, keepdims=True)
        sel = d == mval
        midx = jnp.min(jnp.where(sel, iota, _N), axis=1, keepdims=True)
        cols.append(midx + b * _N)
        d = jnp.where(iota == midx, 1e30, d)
    idx_out[0] = jnp.concatenate(cols, axis=1)


def _knn(aq3, bk3):
    grid = (_B, _N // _TQ)
    return pl.pallas_call(
        _knn_body,
        grid=grid,
        in_specs=[pl.BlockSpec((1, _TQ, 8), lambda b, i: (b, i, 0)),
                  pl.BlockSpec((1, 8, _N), lambda b, i: (b, 0, 0))],
        out_specs=pl.BlockSpec((1, _TQ, _K), lambda b, i: (b, i, 0)),
        out_shape=jax.ShapeDtypeStruct((_B, _N, _K), jnp.int32),
    )(aq3, bk3)


def _sc_gather_call(table, idxr):
    mesh = plsc.VectorSubcoreMesh(core_axis_name="c", subcore_axis_name="s")
    nw = 32
    chunks = _M // (nw * 128)

    @functools.partial(
        pl.kernel,
        out_type=jax.ShapeDtypeStruct((_M, _W), _F32),
        mesh=mesh,
        scratch_types=[pltpu.VMEM((chunks, 128), jnp.int32),
                       pltpu.VMEM((128, _W), _F32),
                       pltpu.SemaphoreType.DMA],
    )
    def gat(t_hbm, idx_hbm, g_hbm, idx_v, rows_v, sem):
        c = lax.axis_index("c")
        s = lax.axis_index("s")
        wid = s * 2 + c
        pltpu.sync_copy(idx_hbm.at[pl.ds(wid * chunks, chunks)], idx_v)

        def body(j, carry):
            pltpu.async_copy(t_hbm.at[idx_v.at[j]], rows_v, sem).wait()
            base = wid * (chunks * 128) + j * 128
            pltpu.sync_copy(rows_v, g_hbm.at[pl.ds(base, 128)])
            return carry

        lax.fori_loop(0, chunks, body, 0)

    return gat(table, idxr)


_TM = 128


def _ext_dir(dirf):
    li = lax.broadcasted_iota(jnp.int32, dirf.shape, 1)
    return jnp.where(li < 3, dirf, jnp.where(li == 3, 1.0, 0.0))


def _mom_body(gx, q, mom):
    g = gx[...].reshape(_TM, _K, 128)
    dirf = (g - q[...][:, None, :]).reshape(_TM * _K, 128)
    d = _ext_dir(dirf)
    part = lax.dot_general(d, d, (((0,), (0,)), ((), ())),
                           preferred_element_type=_F32)

    @pl.when(pl.program_id(0) == 0)
    def _():
        mom[...] = jnp.zeros_like(mom)

    mom[...] += part


def _moments(gcomb, qw):
    grid = (_BN // _TM,)
    return pl.pallas_call(
        _mom_body,
        grid=grid,
        in_specs=[pl.BlockSpec((_TM * _K, 128), lambda i: (i, 2)),
                  pl.BlockSpec((_TM, 128), lambda i: (i, 0))],
        out_specs=pl.BlockSpec((128, 128), lambda i: (0, 0)),
        out_shape=jax.ShapeDtypeStruct((128, 128), _F32),
    )(gcomb, qw)


def _weightnet(dirf, A, gamma, beta, Wb, bb, mom):
    minv = _F32(1.0 / _M)
    d = _ext_dir(dirf)
    mu = jnp.dot(mom[3:4, :] * minv, A, preferred_element_type=_F32)
    qa = jnp.dot(mom * minv, A, preferred_element_type=_F32)
    q = lax.dot_general(A, qa, (((0,), (0,)), ((), ())),
                        preferred_element_type=_F32)
    r = lax.broadcasted_iota(jnp.int32, (8, 8), 0)
    cidx = lax.broadcasted_iota(jnp.int32, (8, 8), 1)
    eh2 = jnp.sum(jnp.where(r == cidx, q, 0.0), axis=0, keepdims=True)
    var = eh2 - mu * mu
    rstd = lax.rsqrt(var + 1e-5)
    h = jnp.dot(d, A, preferred_element_type=_F32)
    hn = (h - mu) * (rstd * gamma) + beta
    hr = jnp.maximum(hn, 0.0)
    return jnp.dot(hr, Wb, preferred_element_type=_F32) + bb


def _lrelu(x):
    return jnp.where(x >= 0, x, 0.1 * x)


def _fuse1_body(gf, gx, y1p, q, W1, b1, W2, b2, A, gam, bet, Wb, bb, mom, out):
    y = y1p[...]
    h0 = _lrelu(gf[...].reshape(_TM, _K, _C) + y[:, None, :]).reshape(_TM * _K, _C)
    h1 = _lrelu(jnp.dot(h0, W1[...], preferred_element_type=_F32) + b1[...])
    h2 = _lrelu(jnp.dot(h1, W2[...], preferred_element_type=_F32) + b2[...])
    qv = q[...]
    g = gx[...].reshape(_TM, _K, 128)
    dirf = (g - qv[:, None, :]).reshape(_TM * _K, 128)
    w = _weightnet(dirf, A[...], gam[...], bet[...], Wb[...], bb[...], mom[...])
    acc = jnp.sum((w * h2).reshape(_TM, _K, _C), axis=1)
    out[...] = jnp.concatenate([acc, qv], axis=1)


def _fuse1(gcomb, y1p, qw, W1, b1r, W2, b2r, A, gamr, betr, Wb, bbr, mom):
    grid = (_BN // _TM,)
    rowk = lambda i: (i, 0)
    const = lambda i: (0, 0)
    return pl.pallas_call(
        _fuse1_body,
        grid=grid,
        in_specs=[pl.BlockSpec((_TM * _K, _C), rowk),
                  pl.BlockSpec((_TM * _K, 128), lambda i: (i, 2)),
                  pl.BlockSpec((_TM, _C), rowk),
                  pl.BlockSpec((_TM, 128), rowk),
                  pl.BlockSpec((_C, _C), const), pl.BlockSpec((1, _C), const),
                  pl.BlockSpec((_C, _C), const), pl.BlockSpec((1, _C), const),
                  pl.BlockSpec((128, 8), const), pl.BlockSpec((1, 8), const),
                  pl.BlockSpec((1, 8), const), pl.BlockSpec((8, _C), const),
                  pl.BlockSpec((1, _C), const), pl.BlockSpec((128, 128), const)],
        out_specs=pl.BlockSpec((_TM, _W), rowk),
        out_shape=jax.ShapeDtypeStruct((_BN, _W), _F32),
    )(gcomb, gcomb, y1p, qw, W1, b1r, W2, b2r, A, gamr, betr, Wb, bbr, mom)


def _fuse2_body(gf, gx, q, A, gam, bet, Wb, bb, mom, out):
    g = gx[...].reshape(_TM, _K, 128)
    dirf = (g - q[...][:, None, :]).reshape(_TM * _K, 128)
    w = _weightnet(dirf, A[...], gam[...], bet[...], Wb[...], bb[...], mom[...])
    out[...] = jnp.sum((w * gf[...]).reshape(_TM, _K, _C), axis=1)


def _fuse2(gcomb, qw, A, gamr, betr, Wb, bbr, mom):
    grid = (_BN // _TM,)
    rowk = lambda i: (i, 0)
    const = lambda i: (0, 0)
    return pl.pallas_call(
        _fuse2_body,
        grid=grid,
        in_specs=[pl.BlockSpec((_TM * _K, _C), rowk),
                  pl.BlockSpec((_TM * _K, 128), lambda i: (i, 2)),
                  pl.BlockSpec((_TM, 128), rowk),
                  pl.BlockSpec((128, 8), const), pl.BlockSpec((1, 8), const),
                  pl.BlockSpec((1, 8), const), pl.BlockSpec((8, _C), const),
                  pl.BlockSpec((1, _C), const), pl.BlockSpec((128, 128), const)],
        out_specs=pl.BlockSpec((_TM, _C), rowk),
        out_shape=jax.ShapeDtypeStruct((_BN, _C), _F32),
    )(gcomb, gcomb, qw, A, gamr, betr, Wb, bbr, mom)


def kernel(pc1, pc2, feature1, feature2, W0, b0, W1, b1, W2, b2,
           wn1_Wa, wn1_ba, wn1_gamma, wn1_beta, wn1_Wb, wn1_bb,
           wn2_Wa, wn2_ba, wn2_gamma, wn2_beta, wn2_Wb, wn2_bb):
    pc1t = pc1.transpose(0, 2, 1).reshape(_BN, 3)
    pc2t = pc2.transpose(0, 2, 1).reshape(_BN, 3)
    f1t = feature1.transpose(0, 2, 1).reshape(_BN, _C)
    f2t = feature2.transpose(0, 2, 1).reshape(_BN, _C)
    pc1p8 = jnp.pad(pc1t, ((0, 0), (0, 5)))
    pc2p8 = jnp.pad(pc2t, ((0, 0), (0, 5)))
    pc1w = jnp.pad(pc1t, ((0, 0), (0, 125)))
    pc2w = jnp.pad(pc2t, ((0, 0), (0, 125)))
    pc1T8 = jnp.pad(pc1.transpose(1, 0, 2).reshape(3, _BN), ((0, 5), (0, 0)))
    pc2T8 = jnp.pad(pc2.transpose(1, 0, 2).reshape(3, _BN), ((0, 5), (0, 0)))

    W0f1 = W0[0:_C]
    W0f2 = W0[_C:2 * _C]
    W0d8 = jnp.pad(W0[2 * _C:], ((0, 5), (0, 0)))
    b0r = b0.reshape(1, _C)

    aq, bk2, bk1, y1p, t1 = _prep(pc1p8, pc2p8, pc1T8, pc2T8, pc2w,
                                  f1t, f2t, W0f1, W0f2, W0d8, b0r)

    aq3 = aq.reshape(_B, _N, 8)
    bk2_3 = bk2.reshape(8, _B, _N).transpose(1, 0, 2)
    bk1_3 = bk1.reshape(8, _B, _N).transpose(1, 0, 2)
    idx1 = _knn(aq3, bk2_3)
    idx2 = _knn(aq3, bk1_3)

    g1 = _sc_gather_call(t1, idx1.reshape(_M // 128, 128))
    mom1 = _moments(g1, pc1w)
    A1 = jnp.concatenate([wn1_Wa, wn1_ba.reshape(1, 8), jnp.zeros((124, 8), _F32)], axis=0)
    x1 = _fuse1(g1, y1p, pc1w,
                W1, b1.reshape(1, _C), W2, b2.reshape(1, _C),
                A1, wn1_gamma.reshape(1, 8), wn1_beta.reshape(1, 8),
                wn1_Wb, wn1_bb.reshape(1, _C), mom1)

    g2 = _sc_gather_call(x1, idx2.reshape(_M // 128, 128))
    mom2 = _moments(g2, pc1w)
    A2 = jnp.concatenate([wn2_Wa, wn2_ba.reshape(1, 8), jnp.zeros((124, 8), _F32)], axis=0)
    out = _fuse2(g2, pc1w,
                 A2, wn2_gamma.reshape(1, 8), wn2_beta.reshape(1, 8),
                 wn2_Wb, wn2_bb.reshape(1, _C), mom2)

    return out.reshape(_B, _N, _C).transpose(0, 2, 1)


# speedup vs baseline: 13.9416x; 1.0563x over previous
import functools

import jax
import jax.numpy as jnp
from jax import lax
from jax.experimental import pallas as pl
from jax.experimental.pallas import tpu as pltpu
from jax.experimental.pallas import tpu_sc as plsc

_B, _N, _C, _K = 2, 4096, 256, 16
_BN = _B * _N
_M = _BN * _K
_W = _C + 128
_F32 = jnp.float32


def _prep_body(pc1p, pc2p, pc1T, pc2T, pc2w, f1, f2, W0f1, W0f2, W0d8, b0,
               aq, bk2, bk1, y1p, t1):
    x1 = pc1p[...]
    x2 = pc2p[...]
    n1 = jnp.sum(x1 * x1, axis=1, keepdims=True)
    li = lax.broadcasted_iota(jnp.int32, x1.shape, 1)
    aq[...] = jnp.where(li < 3, x1, jnp.where(li == 4, n1, 0.0))
    for xt, bk in ((pc1T, bk1), (pc2T, bk2)):
        xv = xt[...]
        nn = (xv[0:1, :] * xv[0:1, :] + xv[1:2, :] * xv[1:2, :]
              + xv[2:3, :] * xv[2:3, :])
        si = lax.broadcasted_iota(jnp.int32, xv.shape, 0)
        bk[...] = jnp.where(si < 3, xv, jnp.where(si == 3, nn, 0.0))
    y1p[...] = (jnp.dot(f1[...], W0f1[...], preferred_element_type=_F32) + b0[...]
                - jnp.dot(x1, W0d8[...], preferred_element_type=_F32))
    feat = (jnp.dot(f2[...], W0f2[...], preferred_element_type=_F32)
            + jnp.dot(x2, W0d8[...], preferred_element_type=_F32))
    t1[...] = jnp.concatenate([feat, pc2w[...]], axis=1)


def _prep(pc1p8, pc2p8, pc1T8, pc2T8, pc2w, f1, f2, W0f1, W0f2, W0d8, b0r):
    T = 512
    grid = (_BN // T,)
    row = lambda i: (i, 0)
    col = lambda i: (0, i)
    const = lambda i: (0, 0)
    return pl.pallas_call(
        _prep_body,
        grid=grid,
        in_specs=[pl.BlockSpec((T, 8), row), pl.BlockSpec((T, 8), row),
                  pl.BlockSpec((8, T), col), pl.BlockSpec((8, T), col),
                  pl.BlockSpec((T, 128), row),
                  pl.BlockSpec((T, _C), row), pl.BlockSpec((T, _C), row),
                  pl.BlockSpec((_C, _C), const), pl.BlockSpec((_C, _C), const),
                  pl.BlockSpec((8, _C), const), pl.BlockSpec((1, _C), const)],
        out_specs=[pl.BlockSpec((T, 8), row), pl.BlockSpec((8, T), col),
                   pl.BlockSpec((8, T), col),
                   pl.BlockSpec((T, _C), row), pl.BlockSpec((T, _W), row)],
        out_shape=[jax.ShapeDtypeStruct((_BN, 8), _F32),
                   jax.ShapeDtypeStruct((8, _BN), _F32),
                   jax.ShapeDtypeStruct((8, _BN), _F32),
                   jax.ShapeDtypeStruct((_BN, _C), _F32),
                   jax.ShapeDtypeStruct((_BN, _W), _F32)],
    )(pc1p8, pc2p8, pc1T8, pc2T8, pc2w, f1, f2, W0f1, W0f2, W0d8, b0r)


_TQ = 256


def _knn_body(aq, bk, idx_out):
    q = aq[0]
    ks = bk[0, 0]
    li = lax.broadcasted_iota(jnp.int32, q.shape, 1)
    si = lax.broadcasted_iota(jnp.int32, ks.shape, 0)
    qm = jnp.where(li < 3, q, 0.0)
    km = jnp.where(si < 3, ks, 0.0)
    s = lax.dot_general(qm, km, (((1,), (0,)), ((), ())),
                        preferred_element_type=_F32)
    d = (q[:, 4:5] + ks[3:4, :]) - 2.0 * s
    b = pl.program_id(1)
    iota = lax.broadcasted_iota(jnp.int32, d.shape, 1)
    cols = []
    for _ in range(_K):
        am = jnp.argmin(d, axis=1)[:, None].astype(jnp.int32)
        cols.append(am + b * _N)
        d = jnp.where(iota == am, 1e30, d)
    idx_out[0, 0] = jnp.concatenate(cols, axis=1)


def _knn(aq3, bks):
    grid = (2, _B, _N // _TQ)
    return pl.pallas_call(
        _knn_body,
        grid=grid,
        in_specs=[pl.BlockSpec((1, _TQ, 8), lambda s, b, i: (b, i, 0)),
                  pl.BlockSpec((1, 1, 8, _N), lambda s, b, i: (s, b, 0, 0))],
        out_specs=pl.BlockSpec((1, 1, _TQ, _K), lambda s, b, i: (s, b, i, 0)),
        out_shape=jax.ShapeDtypeStruct((2, _B, _N, _K), jnp.int32),
    )(aq3, bks)


def _sc_gather_call(table, idxr):
    mesh = plsc.VectorSubcoreMesh(core_axis_name="c", subcore_axis_name="s")
    nw = 32
    chunks = _M // (nw * 128)

    @functools.partial(
        pl.kernel,
        out_type=jax.ShapeDtypeStruct((_M, _W), _F32),
        mesh=mesh,
        scratch_types=[pltpu.VMEM((chunks, 128), jnp.int32),
                       pltpu.VMEM((128, _W), _F32),
                       pltpu.SemaphoreType.DMA],
    )
    def gat(t_hbm, idx_hbm, g_hbm, idx_v, rows_v, sem):
        c = lax.axis_index("c")
        s = lax.axis_index("s")
        wid = s * 2 + c
        pltpu.sync_copy(idx_hbm.at[pl.ds(wid * chunks, chunks)], idx_v)

        def body(j, carry):
            pltpu.async_copy(t_hbm.at[idx_v.at[j]], rows_v, sem).wait()
            base = wid * (chunks * 128) + j * 128
            pltpu.sync_copy(rows_v, g_hbm.at[pl.ds(base, 128)])
            return carry

        lax.fori_loop(0, chunks, body, 0)

    return gat(table, idxr)


_TM = 256


def _ext_dir(dirf):
    li = lax.broadcasted_iota(jnp.int32, dirf.shape, 1)
    return jnp.where(li < 3, dirf, jnp.where(li == 3, 1.0, 0.0))


def _mom_body(gx, q, mom):
    g = gx[...].reshape(_TM, _K, 128)
    dirf = (g - q[...][:, None, :]).reshape(_TM * _K, 128)
    d = _ext_dir(dirf)
    part = lax.dot_general(d, d, (((0,), (0,)), ((), ())),
                           preferred_element_type=_F32)

    @pl.when(pl.program_id(0) == 0)
    def _():
        mom[...] = jnp.zeros_like(mom)

    mom[...] += part


def _moments(gcomb, qw):
    grid = (_BN // _TM,)
    return pl.pallas_call(
        _mom_body,
        grid=grid,
        in_specs=[pl.BlockSpec((_TM * _K, 128), lambda i: (i, 2)),
                  pl.BlockSpec((_TM, 128), lambda i: (i, 0))],
        out_specs=pl.BlockSpec((128, 128), lambda i: (0, 0)),
        out_shape=jax.ShapeDtypeStruct((128, 128), _F32),
    )(gcomb, qw)


def _weightnet(dirf, A, gamma, beta, Wb, bb, mom):
    minv = _F32(1.0 / _M)
    d = _ext_dir(dirf)
    mu = jnp.dot(mom[3:4, :] * minv, A, preferred_element_type=_F32)
    qa = jnp.dot(mom * minv, A, preferred_element_type=_F32)
    q = lax.dot_general(A, qa, (((0,), (0,)), ((), ())),
                        preferred_element_type=_F32)
    r = lax.broadcasted_iota(jnp.int32, (8, 8), 0)
    cidx = lax.broadcasted_iota(jnp.int32, (8, 8), 1)
    eh2 = jnp.sum(jnp.where(r == cidx, q, 0.0), axis=0, keepdims=True)
    var = eh2 - mu * mu
    rstd = lax.rsqrt(var + 1e-5)
    h = jnp.dot(d, A, preferred_element_type=_F32)
    hn = (h - mu) * (rstd * gamma) + beta
    hr = jnp.maximum(hn, 0.0)
    return jnp.dot(hr, Wb, preferred_element_type=_F32) + bb


def _lrelu(x):
    return jnp.where(x >= 0, x, 0.1 * x)


def _fuse1_body(gf, gx, y1p, q, W1, b1, W2, b2, A, gam, bet, Wb, bb, mom, out):
    y = y1p[...]
    h0 = _lrelu(gf[...].reshape(_TM, _K, _C) + y[:, None, :]).reshape(_TM * _K, _C)
    h1 = _lrelu(jnp.dot(h0, W1[...], preferred_element_type=_F32) + b1[...])
    h2 = _lrelu(jnp.dot(h1, W2[...], preferred_element_type=_F32) + b2[...])
    qv = q[...]
    g = gx[...].reshape(_TM, _K, 128)
    dirf = (g - qv[:, None, :]).reshape(_TM * _K, 128)
    w = _weightnet(dirf, A[...], gam[...], bet[...], Wb[...], bb[...], mom[...])
    acc = jnp.sum((w * h2).reshape(_TM, _K, _C), axis=1)
    out[...] = jnp.concatenate([acc, qv], axis=1)


def _fuse1(gcomb, y1p, qw, W1, b1r, W2, b2r, A, gamr, betr, Wb, bbr, mom):
    grid = (_BN // _TM,)
    rowk = lambda i: (i, 0)
    const = lambda i: (0, 0)
    return pl.pallas_call(
        _fuse1_body,
        grid=grid,
        in_specs=[pl.BlockSpec((_TM * _K, _C), rowk),
                  pl.BlockSpec((_TM * _K, 128), lambda i: (i, 2)),
                  pl.BlockSpec((_TM, _C), rowk),
                  pl.BlockSpec((_TM, 128), rowk),
                  pl.BlockSpec((_C, _C), const), pl.BlockSpec((1, _C), const),
                  pl.BlockSpec((_C, _C), const), pl.BlockSpec((1, _C), const),
                  pl.BlockSpec((128, 8), const), pl.BlockSpec((1, 8), const),
                  pl.BlockSpec((1, 8), const), pl.BlockSpec((8, _C), const),
                  pl.BlockSpec((1, _C), const), pl.BlockSpec((128, 128), const)],
        out_specs=pl.BlockSpec((_TM, _W), rowk),
        out_shape=jax.ShapeDtypeStruct((_BN, _W), _F32),
    )(gcomb, gcomb, y1p, qw, W1, b1r, W2, b2r, A, gamr, betr, Wb, bbr, mom)


def _fuse2_body(gf, gx, q, A, gam, bet, Wb, bb, mom, out):
    g = gx[...].reshape(_TM, _K, 128)
    dirf = (g - q[...][:, None, :]).reshape(_TM * _K, 128)
    w = _weightnet(dirf, A[...], gam[...], bet[...], Wb[...], bb[...], mom[...])
    out[...] = jnp.sum((w * gf[...]).reshape(_TM, _K, _C), axis=1)


def _fuse2(gcomb, qw, A, gamr, betr, Wb, bbr, mom):
    grid = (_BN // _TM,)
    rowk = lambda i: (i, 0)
    const = lambda i: (0, 0)
    return pl.pallas_call(
        _fuse2_body,
        grid=grid,
        in_specs=[pl.BlockSpec((_TM * _K, _C), rowk),
                  pl.BlockSpec((_TM * _K, 128), lambda i: (i, 2)),
                  pl.BlockSpec((_TM, 128), rowk),
                  pl.BlockSpec((128, 8), const), pl.BlockSpec((1, 8), const),
                  pl.BlockSpec((1, 8), const), pl.BlockSpec((8, _C), const),
                  pl.BlockSpec((1, _C), const), pl.BlockSpec((128, 128), const)],
        out_specs=pl.BlockSpec((_TM, _C), rowk),
        out_shape=jax.ShapeDtypeStruct((_BN, _C), _F32),
    )(gcomb, gcomb, qw, A, gamr, betr, Wb, bbr, mom)


def kernel(pc1, pc2, feature1, feature2, W0, b0, W1, b1, W2, b2,
           wn1_Wa, wn1_ba, wn1_gamma, wn1_beta, wn1_Wb, wn1_bb,
           wn2_Wa, wn2_ba, wn2_gamma, wn2_beta, wn2_Wb, wn2_bb):
    pc1t = pc1.transpose(0, 2, 1).reshape(_BN, 3)
    pc2t = pc2.transpose(0, 2, 1).reshape(_BN, 3)
    f1t = feature1.transpose(0, 2, 1).reshape(_BN, _C)
    f2t = feature2.transpose(0, 2, 1).reshape(_BN, _C)
    pc1p8 = jnp.pad(pc1t, ((0, 0), (0, 5)))
    pc2p8 = jnp.pad(pc2t, ((0, 0), (0, 5)))
    pc1w = jnp.pad(pc1t, ((0, 0), (0, 125)))
    pc2w = jnp.pad(pc2t, ((0, 0), (0, 125)))
    pc1T8 = jnp.pad(pc1.transpose(1, 0, 2).reshape(3, _BN), ((0, 5), (0, 0)))
    pc2T8 = jnp.pad(pc2.transpose(1, 0, 2).reshape(3, _BN), ((0, 5), (0, 0)))

    W0f1 = W0[0:_C]
    W0f2 = W0[_C:2 * _C]
    W0d8 = jnp.pad(W0[2 * _C:], ((0, 5), (0, 0)))
    b0r = b0.reshape(1, _C)

    aq, bk2, bk1, y1p, t1 = _prep(pc1p8, pc2p8, pc1T8, pc2T8, pc2w,
                                  f1t, f2t, W0f1, W0f2, W0d8, b0r)

    aq3 = aq.reshape(_B, _N, 8)
    bk2_3 = bk2.reshape(8, _B, _N).transpose(1, 0, 2)
    bk1_3 = bk1.reshape(8, _B, _N).transpose(1, 0, 2)
    idx12 = _knn(aq3, jnp.stack([bk2_3, bk1_3]))
    idx1, idx2 = idx12[0], idx12[1]

    g1 = _sc_gather_call(t1, idx1.reshape(_M // 128, 128))
    mom1 = _moments(g1, pc1w)
    A1 = jnp.concatenate([wn1_Wa, wn1_ba.reshape(1, 8), jnp.zeros((124, 8), _F32)], axis=0)
    x1 = _fuse1(g1, y1p, pc1w,
                W1, b1.reshape(1, _C), W2, b2.reshape(1, _C),
                A1, wn1_gamma.reshape(1, 8), wn1_beta.reshape(1, 8),
                wn1_Wb, wn1_bb.reshape(1, _C), mom1)

    g2 = _sc_gather_call(x1, idx2.reshape(_M // 128, 128))
    mom2 = _moments(g2, pc1w)
    A2 = jnp.concatenate([wn2_Wa, wn2_ba.reshape(1, 8), jnp.zeros((124, 8), _F32)], axis=0)
    out = _fuse2(g2, pc1w,
                 A2, wn2_gamma.reshape(1, 8), wn2_beta.reshape(1, 8),
                 wn2_Wb, wn2_bb.reshape(1, _C), mom2)

    return out.reshape(_B, _N, _C).transpose(0, 2, 1)


# speedup vs baseline: 14.2555x; 1.0225x over previous
import functools

import jax
import jax.numpy as jnp
from jax import lax
from jax.experimental import pallas as pl
from jax.experimental.pallas import tpu as pltpu
from jax.experimental.pallas import tpu_sc as plsc

_B, _N, _C, _K = 2, 4096, 256, 16
_BN = _B * _N
_M = _BN * _K
_W = _C + 128
_F32 = jnp.float32


def _prep_body(pc1p, pc2p, pc1T, pc2T, pc2w, f1, f2, W0f1, W0f2, W0d8, b0,
               aq, bk2, bk1, y1p, t1):
    x1 = pc1p[...]
    x2 = pc2p[...]
    n1 = jnp.sum(x1 * x1, axis=1, keepdims=True)
    li = lax.broadcasted_iota(jnp.int32, x1.shape, 1)
    aq[...] = jnp.where(li < 3, x1, jnp.where(li == 4, n1, 0.0))
    for xt, bk in ((pc1T, bk1), (pc2T, bk2)):
        xv = xt[...]
        nn = (xv[0:1, :] * xv[0:1, :] + xv[1:2, :] * xv[1:2, :]
              + xv[2:3, :] * xv[2:3, :])
        si = lax.broadcasted_iota(jnp.int32, xv.shape, 0)
        bk[...] = jnp.where(si < 3, xv, jnp.where(si == 3, nn, 0.0))
    y1p[...] = (jnp.dot(f1[...], W0f1[...], preferred_element_type=_F32) + b0[...]
                - jnp.dot(x1, W0d8[...], preferred_element_type=_F32))
    feat = (jnp.dot(f2[...], W0f2[...], preferred_element_type=_F32)
            + jnp.dot(x2, W0d8[...], preferred_element_type=_F32))
    t1[...] = jnp.concatenate([feat, pc2w[...]], axis=1)


def _prep(pc1p8, pc2p8, pc1T8, pc2T8, pc2w, f1, f2, W0f1, W0f2, W0d8, b0r):
    T = 512
    grid = (_BN // T,)
    row = lambda i: (i, 0)
    col = lambda i: (0, i)
    const = lambda i: (0, 0)
    return pl.pallas_call(
        _prep_body,
        grid=grid,
        in_specs=[pl.BlockSpec((T, 8), row), pl.BlockSpec((T, 8), row),
                  pl.BlockSpec((8, T), col), pl.BlockSpec((8, T), col),
                  pl.BlockSpec((T, 128), row),
                  pl.BlockSpec((T, _C), row), pl.BlockSpec((T, _C), row),
                  pl.BlockSpec((_C, _C), const), pl.BlockSpec((_C, _C), const),
                  pl.BlockSpec((8, _C), const), pl.BlockSpec((1, _C), const)],
        out_specs=[pl.BlockSpec((T, 8), row), pl.BlockSpec((8, T), col),
                   pl.BlockSpec((8, T), col),
                   pl.BlockSpec((T, _C), row), pl.BlockSpec((T, _W), row)],
        out_shape=[jax.ShapeDtypeStruct((_BN, 8), _F32),
                   jax.ShapeDtypeStruct((8, _BN), _F32),
                   jax.ShapeDtypeStruct((8, _BN), _F32),
                   jax.ShapeDtypeStruct((_BN, _C), _F32),
                   jax.ShapeDtypeStruct((_BN, _W), _F32)],
    )(pc1p8, pc2p8, pc1T8, pc2T8, pc2w, f1, f2, W0f1, W0f2, W0d8, b0r)


_TQ = 256


def _knn_body(aq, bk, idx_out):
    q = aq[0]
    ks = bk[0, 0]
    li = lax.broadcasted_iota(jnp.int32, q.shape, 1)
    si = lax.broadcasted_iota(jnp.int32, ks.shape, 0)
    qm = jnp.where(li < 3, q, 0.0)
    km = jnp.where(si < 3, ks, 0.0)
    s = lax.dot_general(qm, km, (((1,), (0,)), ((), ())),
                        preferred_element_type=_F32)
    d = (q[:, 4:5] + ks[3:4, :]) - 2.0 * s
    b = pl.program_id(1)
    iota = lax.broadcasted_iota(jnp.int32, d.shape, 1)
    cols = []
    for _ in range(_K):
        am = jnp.argmin(d, axis=1)[:, None].astype(jnp.int32)
        cols.append(am + b * _N)
        d = jnp.where(iota == am, 1e30, d)
    idx_out[0, 0] = jnp.concatenate(cols, axis=1)


def _knn(aq3, bks):
    grid = (2, _B, _N // _TQ)
    return pl.pallas_call(
        _knn_body,
        grid=grid,
        in_specs=[pl.BlockSpec((1, _TQ, 8), lambda s, b, i: (b, i, 0)),
                  pl.BlockSpec((1, 1, 8, _N), lambda s, b, i: (s, b, 0, 0))],
        out_specs=pl.BlockSpec((1, 1, _TQ, _K), lambda s, b, i: (s, b, i, 0)),
        out_shape=jax.ShapeDtypeStruct((2, _B, _N, _K), jnp.int32),
    )(aq3, bks)


def _sc_gather_call(table, idxr):
    mesh = plsc.VectorSubcoreMesh(core_axis_name="c", subcore_axis_name="s")
    nw = 32
    chunks = _M // (nw * 128)

    @functools.partial(
        pl.kernel,
        out_type=jax.ShapeDtypeStruct((_M, _W), _F32),
        mesh=mesh,
        scratch_types=[pltpu.VMEM((chunks, 128), jnp.int32),
                       pltpu.VMEM((128, _W), _F32),
                       pltpu.VMEM((128, _W), _F32),
                       pltpu.SemaphoreType.DMA,
                       pltpu.SemaphoreType.DMA],
    )
    def gat(t_hbm, idx_hbm, g_hbm, idx_v, bufa, bufb, sema, semb):
        c = lax.axis_index("c")
        s = lax.axis_index("s")
        wid = s * 2 + c
        base = wid * chunks
        pltpu.sync_copy(idx_hbm.at[pl.ds(base, chunks)], idx_v)

        def wait_for(buf, sem):
            pltpu.make_async_copy(t_hbm.at[pl.ds(0, 128)], buf, sem).wait()

        pltpu.async_copy(t_hbm.at[idx_v.at[0]], bufa, sema)

        def outer(i, carry):
            g = i * 2
            pltpu.async_copy(t_hbm.at[idx_v.at[g + 1]], bufb, semb)
            wait_for(bufa, sema)
            pltpu.sync_copy(bufa, g_hbm.at[pl.ds((base + g) * 128, 128)])

            @pl.when(g + 2 < chunks)
            def _():
                pltpu.async_copy(t_hbm.at[idx_v.at[g + 2]], bufa, sema)

            wait_for(bufb, semb)
            pltpu.sync_copy(bufb, g_hbm.at[pl.ds((base + g + 1) * 128, 128)])
            return carry

        lax.fori_loop(0, chunks // 2, outer, 0)

    return gat(table, idxr)


_TM = 256


def _ext_dir(dirf):
    li = lax.broadcasted_iota(jnp.int32, dirf.shape, 1)
    return jnp.where(li < 3, dirf, jnp.where(li == 3, 1.0, 0.0))


def _mom_body(gx, q, mom):
    g = gx[...].reshape(_TM, _K, 128)
    dirf = (g - q[...][:, None, :]).reshape(_TM * _K, 128)
    d = _ext_dir(dirf)
    part = lax.dot_general(d, d, (((0,), (0,)), ((), ())),
                           preferred_element_type=_F32)

    @pl.when(pl.program_id(0) == 0)
    def _():
        mom[...] = jnp.zeros_like(mom)

    mom[...] += part


def _moments(gcomb, qw):
    grid = (_BN // _TM,)
    return pl.pallas_call(
        _mom_body,
        grid=grid,
        in_specs=[pl.BlockSpec((_TM * _K, 128), lambda i: (i, 2)),
                  pl.BlockSpec((_TM, 128), lambda i: (i, 0))],
        out_specs=pl.BlockSpec((128, 128), lambda i: (0, 0)),
        out_shape=jax.ShapeDtypeStruct((128, 128), _F32),
    )(gcomb, qw)


def _weightnet(dirf, A, gamma, beta, Wb, bb, mom):
    minv = _F32(1.0 / _M)
    d = _ext_dir(dirf)
    mu = jnp.dot(mom[3:4, :] * minv, A, preferred_element_type=_F32)
    qa = jnp.dot(mom * minv, A, preferred_element_type=_F32)
    q = lax.dot_general(A, qa, (((0,), (0,)), ((), ())),
                        preferred_element_type=_F32)
    r = lax.broadcasted_iota(jnp.int32, (8, 8), 0)
    cidx = lax.broadcasted_iota(jnp.int32, (8, 8), 1)
    eh2 = jnp.sum(jnp.where(r == cidx, q, 0.0), axis=0, keepdims=True)
    var = eh2 - mu * mu
    rstd = lax.rsqrt(var + 1e-5)
    h = jnp.dot(d, A, preferred_element_type=_F32)
    hn = (h - mu) * (rstd * gamma) + beta
    hr = jnp.maximum(hn, 0.0)
    return jnp.dot(hr, Wb, preferred_element_type=_F32) + bb


def _lrelu(x):
    return jnp.where(x >= 0, x, 0.1 * x)


def _fuse1_body(gf, gx, y1p, q, W1, b1, W2, b2, A, gam, bet, Wb, bb, mom, out):
    y = y1p[...]
    h0 = _lrelu(gf[...].reshape(_TM, _K, _C) + y[:, None, :]).reshape(_TM * _K, _C)
    h1 = _lrelu(jnp.dot(h0, W1[...], preferred_element_type=_F32) + b1[...])
    h2 = _lrelu(jnp.dot(h1, W2[...], preferred_element_type=_F32) + b2[...])
    qv = q[...]
    g = gx[...].reshape(_TM, _K, 128)
    dirf = (g - qv[:, None, :]).reshape(_TM * _K, 128)
    w = _weightnet(dirf, A[...], gam[...], bet[...], Wb[...], bb[...], mom[...])
    acc = jnp.sum((w * h2).reshape(_TM, _K, _C), axis=1)
    out[...] = jnp.concatenate([acc, qv], axis=1)


def _fuse1(gcomb, y1p, qw, W1, b1r, W2, b2r, A, gamr, betr, Wb, bbr, mom):
    grid = (_BN // _TM,)
    rowk = lambda i: (i, 0)
    const = lambda i: (0, 0)
    return pl.pallas_call(
        _fuse1_body,
        grid=grid,
        in_specs=[pl.BlockSpec((_TM * _K, _C), rowk),
                  pl.BlockSpec((_TM * _K, 128), lambda i: (i, 2)),
                  pl.BlockSpec((_TM, _C), rowk),
                  pl.BlockSpec((_TM, 128), rowk),
                  pl.BlockSpec((_C, _C), const), pl.BlockSpec((1, _C), const),
                  pl.BlockSpec((_C, _C), const), pl.BlockSpec((1, _C), const),
                  pl.BlockSpec((128, 8), const), pl.BlockSpec((1, 8), const),
                  pl.BlockSpec((1, 8), const), pl.BlockSpec((8, _C), const),
                  pl.BlockSpec((1, _C), const), pl.BlockSpec((128, 128), const)],
        out_specs=pl.BlockSpec((_TM, _W), rowk),
        out_shape=jax.ShapeDtypeStruct((_BN, _W), _F32),
    )(gcomb, gcomb, y1p, qw, W1, b1r, W2, b2r, A, gamr, betr, Wb, bbr, mom)


def _fuse2_body(gf, gx, q, A, gam, bet, Wb, bb, mom, out):
    g = gx[...].reshape(_TM, _K, 128)
    dirf = (g - q[...][:, None, :]).reshape(_TM * _K, 128)
    w = _weightnet(dirf, A[...], gam[...], bet[...], Wb[...], bb[...], mom[...])
    out[...] = jnp.sum((w * gf[...]).reshape(_TM, _K, _C), axis=1)


def _fuse2(gcomb, qw, A, gamr, betr, Wb, bbr, mom):
    grid = (_BN // _TM,)
    rowk = lambda i: (i, 0)
    const = lambda i: (0, 0)
    return pl.pallas_call(
        _fuse2_body,
        grid=grid,
        in_specs=[pl.BlockSpec((_TM * _K, _C), rowk),
                  pl.BlockSpec((_TM * _K, 128), lambda i: (i, 2)),
                  pl.BlockSpec((_TM, 128), rowk),
                  pl.BlockSpec((128, 8), const), pl.BlockSpec((1, 8), const),
                  pl.BlockSpec((1, 8), const), pl.BlockSpec((8, _C), const),
                  pl.BlockSpec((1, _C), const), pl.BlockSpec((128, 128), const)],
        out_specs=pl.BlockSpec((_TM, _C), rowk),
        out_shape=jax.ShapeDtypeStruct((_BN, _C), _F32),
    )(gcomb, gcomb, qw, A, gamr, betr, Wb, bbr, mom)


def kernel(pc1, pc2, feature1, feature2, W0, b0, W1, b1, W2, b2,
           wn1_Wa, wn1_ba, wn1_gamma, wn1_beta, wn1_Wb, wn1_bb,
           wn2_Wa, wn2_ba, wn2_gamma, wn2_beta, wn2_Wb, wn2_bb):
    pc1t = pc1.transpose(0, 2, 1).reshape(_BN, 3)
    pc2t = pc2.transpose(0, 2, 1).reshape(_BN, 3)
    f1t = feature1.transpose(0, 2, 1).reshape(_BN, _C)
    f2t = feature2.transpose(0, 2, 1).reshape(_BN, _C)
    pc1p8 = jnp.pad(pc1t, ((0, 0), (0, 5)))
    pc2p8 = jnp.pad(pc2t, ((0, 0), (0, 5)))
    pc1w = jnp.pad(pc1t, ((0, 0), (0, 125)))
    pc2w = jnp.pad(pc2t, ((0, 0), (0, 125)))
    pc1T8 = jnp.pad(pc1.transpose(1, 0, 2).reshape(3, _BN), ((0, 5), (0, 0)))
    pc2T8 = jnp.pad(pc2.transpose(1, 0, 2).reshape(3, _BN), ((0, 5), (0, 0)))

    W0f1 = W0[0:_C]
    W0f2 = W0[_C:2 * _C]
    W0d8 = jnp.pad(W0[2 * _C:], ((0, 5), (0, 0)))
    b0r = b0.reshape(1, _C)

    aq, bk2, bk1, y1p, t1 = _prep(pc1p8, pc2p8, pc1T8, pc2T8, pc2w,
                                  f1t, f2t, W0f1, W0f2, W0d8, b0r)

    aq3 = aq.reshape(_B, _N, 8)
    bk2_3 = bk2.reshape(8, _B, _N).transpose(1, 0, 2)
    bk1_3 = bk1.reshape(8, _B, _N).transpose(1, 0, 2)
    idx12 = _knn(aq3, jnp.stack([bk2_3, bk1_3]))
    idx1, idx2 = idx12[0], idx12[1]

    g1 = _sc_gather_call(t1, idx1.reshape(_M // 128, 128))
    mom1 = _moments(g1, pc1w)
    A1 = jnp.concatenate([wn1_Wa, wn1_ba.reshape(1, 8), jnp.zeros((124, 8), _F32)], axis=0)
    x1 = _fuse1(g1, y1p, pc1w,
                W1, b1.reshape(1, _C), W2, b2.reshape(1, _C),
                A1, wn1_gamma.reshape(1, 8), wn1_beta.reshape(1, 8),
                wn1_Wb, wn1_bb.reshape(1, _C), mom1)

    g2 = _sc_gather_call(x1, idx2.reshape(_M // 128, 128))
    mom2 = _moments(g2, pc1w)
    A2 = jnp.concatenate([wn2_Wa, wn2_ba.reshape(1, 8), jnp.zeros((124, 8), _F32)], axis=0)
    out = _fuse2(g2, pc1w,
                 A2, wn2_gamma.reshape(1, 8), wn2_beta.reshape(1, 8),
                 wn2_Wb, wn2_bb.reshape(1, _C), mom2)

    return out.reshape(_B, _N, _C).transpose(0, 2, 1)
